# baseline (speedup 1.0000x reference)
import functools

import jax
import jax.numpy as jnp
import numpy as np
from jax import lax
from jax.experimental import pallas as pl
from jax.experimental.pallas import tpu as pltpu
from jax.experimental.pallas import tpu_sc as plsc

_R = 128
_N = 32768
_K = 256
_L = 16
_NC = 2
_NS = 16
_NW = _NC * _NS
_ROWS_PER_W = _R // _NW
_SLICES = _N // _L
_UNROLL = 16
_NB = 256
_TOPBIT = np.int32(-(2 ** 31))

_FILTER = np.float32(0.75)
_MAGNIFY = np.float32(1.25)


def _ordinal_key(x):
    b = plsc.bitcast(x, jnp.int32)
    return b ^ ((b >> 31) | _TOPBIT)


def _rank_body(y_hbm, out_hbm, row_v, key_v, hist_v):
    cid = lax.axis_index("c")
    sid = lax.axis_index("s")
    wid = sid * _NC + cid

    iota = lax.iota(jnp.int32, _L)
    zero_v = iota * np.int32(0)
    ones_v = zero_v + np.int32(1)
    lane_base = iota * np.int32(_NB)

    def per_row(r, _):
        row = wid * _ROWS_PER_W + r
        pltpu.sync_copy(y_hbm.at[row], row_v)

        def key_pass(i, carry):
            for u in range(_UNROLL):
                off = (i * _UNROLL + u) * _L
                key_v[pl.ds(off, _L)] = _ordinal_key(row_v[pl.ds(off, _L)])
            return carry
        lax.fori_loop(0, _SLICES // _UNROLL, key_pass, np.int32(0))

        prefix = wid * np.int32(0)
        k_rem = prefix + np.int32(_K)
        for p in range(4):
            shift = 24 - 8 * p

            def zero_pass(i, carry):
                for u in range(_UNROLL):
                    off = (i * _UNROLL + u) * _L
                    hist_v[pl.ds(off, _L)] = zero_v
                return carry
            lax.fori_loop(0, (_L * _NB) // (_L * _UNROLL), zero_pass,
                          np.int32(0))

            if p == 0:
                def hist_pass(i, carry):
                    for u in range(_UNROLL):
                        off = (i * _UNROLL + u) * _L
                        ks = key_v[pl.ds(off, _L)]
                        bucket = lax.shift_right_logical(
                            ks, np.int32(shift)) & np.int32(0xFF)
                        plsc.addupdate_scatter(
                            hist_v, [lane_base + bucket], ones_v)
                    return carry
            else:
                def hist_pass(i, carry, _shift=shift, _pref=prefix):
                    for u in range(_UNROLL):
                        off = (i * _UNROLL + u) * _L
                        ks = key_v[pl.ds(off, _L)]
                        active = lax.shift_right_logical(
                            ks ^ _pref, np.int32(_shift + 8)) == 0
                        bucket = lax.shift_right_logical(
                            ks, np.int32(_shift)) & np.int32(0xFF)
                        plsc.addupdate_scatter(
                            hist_v, [lane_base + bucket], ones_v,
                            mask=active)
                    return carry
            lax.fori_loop(0, _SLICES // _UNROLL, hist_pass, np.int32(0))

            def find_chunk(c, carry):
                b_cur, sub_cur, above = carry
                cc = 15 - c
                base = cc * _L
                tot = zero_v
                for lane in range(_L):
                    tot = tot + hist_v[pl.ds(lane * _NB + base, _L)]
                csum = plsc.cumsum(tot)
                tot_sum = jnp.sum(tot)
                s_incl = tot_sum - csum + tot
                hi = above + s_incl
                m = (hi >= k_rem) & (hi - tot < k_rem)
                mi = m.astype(jnp.int32)
                hit = jnp.sum(mi)
                b_loc = jnp.sum(mi * iota)
                sub_v = jnp.sum(jnp.where(m, hi - tot, zero_v))
                b_new = jnp.where(hit > 0, cc * _L + b_loc, b_cur)
                sub_new = jnp.where(hit > 0, sub_v, sub_cur)
                return (b_new, sub_new, above + tot_sum)

            zero_s = prefix * np.int32(0)
            b_fin, sub_fin, _unused = lax.fori_loop(
                0, _L, find_chunk, (zero_s, zero_s, zero_s))
            prefix = prefix | lax.shift_left(b_fin, np.int32(shift))
            k_rem = k_rem - sub_fin

        t_s = prefix ^ _TOPBIT

        def scale_pass(i, carry):
            for u in range(_UNROLL):
                off = (i * _UNROLL + u) * _L
                x = row_v[pl.ds(off, _L)]
                ks = key_v[pl.ds(off, _L)]
                m = (ks ^ _TOPBIT) < t_s
                row_v[pl.ds(off, _L)] = jnp.where(
                    m, x * _FILTER, x * _MAGNIFY)
            return carry
        lax.fori_loop(0, _SLICES // _UNROLL, scale_pass, np.int32(0))

        pltpu.sync_copy(row_v, out_hbm.at[row])
        return np.int32(0)

    lax.fori_loop(0, _ROWS_PER_W, per_row, np.int32(0))


_rank_sc = functools.partial(
    pl.kernel,
    out_type=jax.ShapeDtypeStruct((_R, _N), jnp.float32),
    mesh=plsc.VectorSubcoreMesh(core_axis_name="c", subcore_axis_name="s"),
    scratch_types=[
        pltpu.VMEM((_N,), jnp.float32),
        pltpu.VMEM((_N,), jnp.int32),
        pltpu.VMEM((_L * _NB,), jnp.int32),
    ],
    compiler_params=pltpu.CompilerParams(needs_layout_passes=False),
)(_rank_body)


def kernel(y):
    return _rank_sc(y)


# speedup vs baseline: 2.9022x; 2.9022x over previous
import functools

import jax
import jax.numpy as jnp
import numpy as np
from jax import lax
from jax.experimental import pallas as pl
from jax.experimental.pallas import tpu as pltpu
from jax.experimental.pallas import tpu_sc as plsc

_R = 128
_N = 32768
_K = 256
_L = 16
_NC = 2
_NS = 16
_NW = _NC * _NS
_ROWS_PER_W = _R // _NW
_SLICES = _N // _L
_UNROLL = 16
_NB = 256
_TOPBIT = np.int32(-(2 ** 31))

_FILTER = np.float32(0.75)
_MAGNIFY = np.float32(1.25)


def _ordinal_key(x):
    b = plsc.bitcast(x, jnp.int32)
    return b ^ ((b >> 31) | _TOPBIT)


def _rank_body(y_hbm, out_hbm, row_v, key_v, hist_v):
    cid = lax.axis_index("c")
    sid = lax.axis_index("s")
    wid = sid * _NC + cid

    iota = lax.iota(jnp.int32, _L)
    zero_v = iota * np.int32(0)
    ones_v = zero_v + np.int32(1)
    lane_base = iota * np.int32(_NB)

    def per_row(r, _):
        row = wid * _ROWS_PER_W + r
        pltpu.sync_copy(y_hbm.at[row], row_v)

        @plsc.parallel_loop(0, _SLICES, 1, unroll=_UNROLL)
        def key_pass(i):
            off = i * _L
            key_v[pl.ds(off, _L)] = _ordinal_key(row_v[pl.ds(off, _L)])

        prefix = wid * np.int32(0)
        k_rem = prefix + np.int32(_K)
        for p in range(4):
            shift = 24 - 8 * p

            @plsc.parallel_loop(0, (_L * _NB) // _L, 1, unroll=_UNROLL)
            def zero_pass(i):
                hist_v[pl.ds(i * _L, _L)] = zero_v

            if p == 0:
                @plsc.parallel_loop(0, _SLICES, 1, unroll=_UNROLL)
                def hist_pass(i):
                    ks = key_v[pl.ds(i * _L, _L)]
                    bucket = lax.shift_right_logical(
                        ks, np.int32(shift)) & np.int32(0xFF)
                    plsc.addupdate_scatter(
                        hist_v, [lane_base + bucket], ones_v)
            else:
                _shift, _pref = shift, prefix

                @plsc.parallel_loop(0, _SLICES, 1, unroll=_UNROLL)
                def hist_pass(i):
                    ks = key_v[pl.ds(i * _L, _L)]
                    active = lax.shift_right_logical(
                        ks ^ _pref, np.int32(_shift + 8)) == 0
                    bucket = lax.shift_right_logical(
                        ks, np.int32(_shift)) & np.int32(0xFF)
                    plsc.addupdate_scatter(
                        hist_v, [lane_base + bucket], ones_v, mask=active)

            def find_chunk(c, carry):
                b_cur, sub_cur, above = carry
                cc = 15 - c
                base = cc * _L
                tot = zero_v
                for lane in range(_L):
                    tot = tot + hist_v[pl.ds(lane * _NB + base, _L)]
                csum = plsc.cumsum(tot)
                tot_sum = jnp.sum(tot)
                s_incl = tot_sum - csum + tot
                hi = above + s_incl
                m = (hi >= k_rem) & (hi - tot < k_rem)
                mi = m.astype(jnp.int32)
                hit = jnp.sum(mi)
                b_loc = jnp.sum(mi * iota)
                sub_v = jnp.sum(jnp.where(m, hi - tot, zero_v))
                b_new = jnp.where(hit > 0, cc * _L + b_loc, b_cur)
                sub_new = jnp.where(hit > 0, sub_v, sub_cur)
                return (b_new, sub_new, above + tot_sum)

            zero_s = prefix * np.int32(0)
            b_fin, sub_fin, _unused = lax.fori_loop(
                0, _L, find_chunk, (zero_s, zero_s, zero_s))
            prefix = prefix | lax.shift_left(b_fin, np.int32(shift))
            k_rem = k_rem - sub_fin

        t_s = prefix ^ _TOPBIT

        @plsc.parallel_loop(0, _SLICES, 1, unroll=_UNROLL)
        def scale_pass(i):
            off = i * _L
            x = row_v[pl.ds(off, _L)]
            ks = key_v[pl.ds(off, _L)]
            m = (ks ^ _TOPBIT) < t_s
            row_v[pl.ds(off, _L)] = jnp.where(m, x * _FILTER, x * _MAGNIFY)

        pltpu.sync_copy(row_v, out_hbm.at[row])
        return np.int32(0)

    lax.fori_loop(0, _ROWS_PER_W, per_row, np.int32(0))


_rank_sc = functools.partial(
    pl.kernel,
    out_type=jax.ShapeDtypeStruct((_R, _N), jnp.float32),
    mesh=plsc.VectorSubcoreMesh(core_axis_name="c", subcore_axis_name="s"),
    scratch_types=[
        pltpu.VMEM((_N,), jnp.float32),
        pltpu.VMEM((_N,), jnp.int32),
        pltpu.VMEM((_L * _NB,), jnp.int32),
    ],
    compiler_params=pltpu.CompilerParams(needs_layout_passes=False),
)(_rank_body)


def kernel(y):
    return _rank_sc(y)


# speedup vs baseline: 2.9132x; 1.0038x over previous
import functools

import jax
import jax.numpy as jnp
import numpy as np
from jax import lax
from jax.experimental import pallas as pl
from jax.experimental.pallas import tpu as pltpu
from jax.experimental.pallas import tpu_sc as plsc

_R = 128
_N = 32768
_K = 256
_L = 16
_NC = 2
_NS = 16
_NW = _NC * _NS
_ROWS_PER_W = _R // _NW
_SLICES = _N // _L
_UNROLL = 16
_NB = 256
_TOPBIT = np.int32(-(2 ** 31))
_MAXPOS = np.int32(0x7FFFFFFF)

_FILTER = np.float32(0.75)
_MAGNIFY = np.float32(1.25)


def _rank_body(y_hbm, out_hbm, buf0, buf1, buf2, hist_v, merged_v,
               in_sem, out_sem):
    cid = lax.axis_index("c")
    sid = lax.axis_index("s")
    wid = sid * _NC + cid
    row0 = wid * _ROWS_PER_W

    iota = lax.iota(jnp.int32, _L)
    zero_v = iota * np.int32(0)
    ones_v = zero_v + np.int32(1)
    lane_base = iota * np.int32(_NB)
    bufs = [buf0, buf1, buf2]

    @plsc.parallel_loop(0, (_L * _NB) // _L, 1, unroll=_UNROLL)
    def zero_pass(i):
        hist_v[pl.ds(i * _L, _L)] = zero_v

    pltpu.async_copy(y_hbm.at[row0], buf0, in_sem)

    for r in range(_ROWS_PER_W):
        b = bufs[r % 3]
        nxt = bufs[(r + 1) % 3]
        if r + 1 < _ROWS_PER_W:
            if r - 2 >= 0:
                pltpu.make_async_copy(
                    nxt, out_hbm.at[row0 + r - 2], out_sem).wait()
            pltpu.async_copy(y_hbm.at[row0 + r + 1], nxt, in_sem)
        pltpu.make_async_copy(y_hbm.at[row0 + r], b, in_sem).wait()

        @plsc.parallel_loop(0, _SLICES, 1, unroll=_UNROLL)
        def hist0_pass(i):
            off = i * _L
            bi = plsc.bitcast(b[pl.ds(off, _L)], jnp.int32)
            ku = bi ^ ((bi >> 31) | _TOPBIT)
            ku = ku + (ku == _MAXPOS).astype(jnp.int32)
            b[pl.ds(off, _L)] = plsc.bitcast(ku, jnp.float32)
            bucket = lax.shift_right_logical(ku, np.int32(24))
            plsc.addupdate_scatter(hist_v, [lane_base + bucket], ones_v)

        prefix = wid * np.int32(0)
        k_rem = prefix + np.int32(_K)
        for p in range(4):
            shift = 24 - 8 * p
            if p > 0:
                _shift, _pref = shift, prefix

                @plsc.parallel_loop(0, _SLICES, 1, unroll=_UNROLL)
                def hist_pass(i):
                    ku = plsc.bitcast(b[pl.ds(i * _L, _L)], jnp.int32)
                    active = lax.shift_right_logical(
                        ku ^ _pref, np.int32(_shift + 8)) == 0
                    bucket = lax.shift_right_logical(
                        ku, np.int32(_shift)) & np.int32(0xFF)
                    plsc.addupdate_scatter(
                        hist_v, [lane_base + bucket], ones_v, mask=active)

            @plsc.parallel_loop(0, _NB // _L, 1, unroll=2)
            def merge_pass(c):
                base = c * _L
                tot = zero_v
                for lane in range(_L):
                    tot = tot + hist_v[pl.ds(lane * _NB + base, _L)]
                    hist_v[pl.ds(lane * _NB + base, _L)] = zero_v
                merged_v[pl.ds(base, _L)] = tot

            csums = zero_v
            for j in range(_L):
                csums = csums + plsc.load_gather(
                    merged_v, [iota * np.int32(_L) + np.int32(j)])

            cs = plsc.cumsum(csums)
            tot_all = jnp.sum(csums)
            hi = tot_all - cs + csums
            m = (hi >= k_rem) & (hi - csums < k_rem)
            mi = m.astype(jnp.int32)
            chunk = jnp.sum(mi * iota)
            above = jnp.sum(jnp.where(m, hi - csums, zero_v))

            tot_c = merged_v[pl.ds(chunk * _L, _L)]
            cs2 = plsc.cumsum(tot_c)
            hi2 = above + (jnp.sum(tot_c) - cs2 + tot_c)
            m2 = (hi2 >= k_rem) & (hi2 - tot_c < k_rem)
            mi2 = m2.astype(jnp.int32)
            b_loc = jnp.sum(mi2 * iota)
            sub = jnp.sum(jnp.where(m2, hi2 - tot_c, zero_v))

            prefix = prefix | lax.shift_left(chunk * _L + b_loc,
                                             np.int32(shift))
            k_rem = k_rem - sub

        t_s = prefix ^ _TOPBIT

        @plsc.parallel_loop(0, _SLICES, 1, unroll=_UNROLL)
        def scale_pass(i):
            off = i * _L
            ku = plsc.bitcast(b[pl.ds(off, _L)], jnp.int32)
            m = (ku ^ _TOPBIT) < t_s
            bi = jnp.where(ku < 0, ku ^ _TOPBIT, ~ku)
            x = plsc.bitcast(bi, jnp.float32)
            b[pl.ds(off, _L)] = jnp.where(m, x * _FILTER, x * _MAGNIFY)

        pltpu.async_copy(b, out_hbm.at[row0 + r], out_sem)

    for r in range(max(0, _ROWS_PER_W - 3), _ROWS_PER_W):
        pltpu.make_async_copy(
            bufs[r % 3], out_hbm.at[row0 + r], out_sem).wait()


_rank_sc = functools.partial(
    pl.kernel,
    out_type=jax.ShapeDtypeStruct((_R, _N), jnp.float32),
    mesh=plsc.VectorSubcoreMesh(core_axis_name="c", subcore_axis_name="s"),
    scratch_types=[
        pltpu.VMEM((_N,), jnp.float32),
        pltpu.VMEM((_N,), jnp.float32),
        pltpu.VMEM((_N,), jnp.float32),
        pltpu.VMEM((_L * _NB,), jnp.int32),
        pltpu.VMEM((_NB,), jnp.int32),
        pltpu.SemaphoreType.DMA,
        pltpu.SemaphoreType.DMA,
    ],
    compiler_params=pltpu.CompilerParams(needs_layout_passes=False),
)(_rank_body)


def kernel(y):
    return _rank_sc(y)


# speedup vs baseline: 3.5686x; 1.2250x over previous
import functools

import jax
import jax.numpy as jnp
import numpy as np
from jax import lax
from jax.experimental import pallas as pl
from jax.experimental.pallas import tpu as pltpu
from jax.experimental.pallas import tpu_sc as plsc

_R = 128
_N = 32768
_K = 256
_L = 16
_NC = 2
_NS = 16
_NW = _NC * _NS
_ROWS_PER_W = _R // _NW
_SLICES = _N // _L
_UNROLL = 16
_NB = 256
_TOPBIT = np.int32(-(2 ** 31))
_MAXPOS = np.int32(0x7FFFFFFF)

_FILTER = np.float32(0.75)
_MAGNIFY = np.float32(1.25)


def _rank_body(y_hbm, out_hbm, buf0, buf1, buf2, hist_v, merged_v,
               in_sem, out_sem):
    cid = lax.axis_index("c")
    sid = lax.axis_index("s")
    wid = sid * _NC + cid
    row0 = wid * _ROWS_PER_W

    iota = lax.iota(jnp.int32, _L)
    zero_v = iota * np.int32(0)
    ones_v = zero_v + np.int32(1)
    bufs = [buf0, buf1, buf2]

    @plsc.parallel_loop(0, (_L * _NB) // _L, 1, unroll=_UNROLL)
    def zero_pass(i):
        hist_v[pl.ds(i * _L, _L)] = zero_v

    pltpu.async_copy(y_hbm.at[row0], buf0, in_sem)

    for r in range(_ROWS_PER_W):
        b = bufs[r % 3]
        nxt = bufs[(r + 1) % 3]
        if r + 1 < _ROWS_PER_W:
            if r - 2 >= 0:
                pltpu.make_async_copy(
                    nxt, out_hbm.at[row0 + r - 2], out_sem).wait()
            pltpu.async_copy(y_hbm.at[row0 + r + 1], nxt, in_sem)
        pltpu.make_async_copy(y_hbm.at[row0 + r], b, in_sem).wait()

        @plsc.parallel_loop(0, _SLICES, 1, unroll=_UNROLL)
        def hist0_pass(i):
            off = i * _L
            bi = plsc.bitcast(b[pl.ds(off, _L)], jnp.int32)
            ku = bi ^ ((bi >> 31) | _TOPBIT)
            ku = ku + (ku == _MAXPOS).astype(jnp.int32)
            b[pl.ds(off, _L)] = plsc.bitcast(ku, jnp.float32)
            b16 = lax.shift_right_logical(ku, np.int32(20)) & np.int32(0xFF0)
            plsc.addupdate_scatter(hist_v, [b16 | iota], ones_v)

        prefix = wid * np.int32(0)
        k_rem = prefix + np.int32(_K)
        for p in range(4):
            shift = 24 - 8 * p
            if p > 0:
                _shift, _pref = shift, prefix

                @plsc.parallel_loop(0, _SLICES, 1, unroll=_UNROLL)
                def hist_pass(i):
                    ku = plsc.bitcast(b[pl.ds(i * _L, _L)], jnp.int32)
                    active = lax.shift_right_logical(
                        ku ^ _pref, np.int32(_shift + 8)) == 0
                    if _shift >= 4:
                        b16 = lax.shift_right_logical(
                            ku, np.int32(_shift - 4)) & np.int32(0xFF0)
                    else:
                        b16 = lax.shift_left(
                            ku, np.int32(4 - _shift)) & np.int32(0xFF0)
                    plsc.addupdate_scatter(
                        hist_v, [b16 | iota], ones_v, mask=active)

            @plsc.parallel_loop(0, _L, 1, unroll=2)
            def merge_pass(c):
                acc = zero_v
                for q in range(_L):
                    acc = acc + hist_v[pl.ds(c * _NB + q * _L, _L)]
                tot = jnp.sum(acc)
                plsc.store_scatter(merged_v, [zero_v + c], zero_v + tot,
                                   mask=iota == 0)

            csums = merged_v[pl.ds(0, _L)]

            cs = plsc.cumsum(csums)
            tot_all = jnp.sum(csums)
            hi = tot_all - cs + csums
            m = (hi >= k_rem) & (hi - csums < k_rem)
            mi = m.astype(jnp.int32)
            chunk = jnp.sum(mi * iota)
            above = jnp.sum(jnp.where(m, hi - csums, zero_v))

            @plsc.parallel_loop(0, _L, 1, unroll=4)
            def within_pass(j):
                vb = hist_v[pl.ds((chunk * _L + j) * _L, _L)]
                tb = jnp.sum(vb)
                plsc.store_scatter(merged_v, [zero_v + j], zero_v + tb,
                                   mask=iota == 0)

            tot_c = merged_v[pl.ds(0, _L)]
            cs2 = plsc.cumsum(tot_c)
            hi2 = above + (jnp.sum(tot_c) - cs2 + tot_c)
            m2 = (hi2 >= k_rem) & (hi2 - tot_c < k_rem)
            mi2 = m2.astype(jnp.int32)
            b_loc = jnp.sum(mi2 * iota)
            sub = jnp.sum(jnp.where(m2, hi2 - tot_c, zero_v))

            prefix = prefix | lax.shift_left(chunk * _L + b_loc,
                                             np.int32(shift))
            k_rem = k_rem - sub

            @plsc.parallel_loop(0, (_L * _NB) // _L, 1, unroll=_UNROLL)
            def rezero_pass(i):
                hist_v[pl.ds(i * _L, _L)] = zero_v

        t_s = prefix ^ _TOPBIT

        @plsc.parallel_loop(0, _SLICES, 1, unroll=_UNROLL)
        def scale_pass(i):
            off = i * _L
            ku = plsc.bitcast(b[pl.ds(off, _L)], jnp.int32)
            m = (ku ^ _TOPBIT) < t_s
            bi = jnp.where(ku < 0, ku ^ _TOPBIT, ~ku)
            x = plsc.bitcast(bi, jnp.float32)
            b[pl.ds(off, _L)] = jnp.where(m, x * _FILTER, x * _MAGNIFY)

        pltpu.async_copy(b, out_hbm.at[row0 + r], out_sem)

    for r in range(max(0, _ROWS_PER_W - 3), _ROWS_PER_W):
        pltpu.make_async_copy(
            bufs[r % 3], out_hbm.at[row0 + r], out_sem).wait()


_rank_sc = functools.partial(
    pl.kernel,
    out_type=jax.ShapeDtypeStruct((_R, _N), jnp.float32),
    mesh=plsc.VectorSubcoreMesh(core_axis_name="c", subcore_axis_name="s"),
    scratch_types=[
        pltpu.VMEM((_N,), jnp.float32),
        pltpu.VMEM((_N,), jnp.float32),
        pltpu.VMEM((_N,), jnp.float32),
        pltpu.VMEM((_L * _NB,), jnp.int32),
        pltpu.VMEM((_L,), jnp.int32),
        pltpu.SemaphoreType.DMA,
        pltpu.SemaphoreType.DMA,
    ],
    compiler_params=pltpu.CompilerParams(needs_layout_passes=False),
)(_rank_body)


def kernel(y):
    return _rank_sc(y)
